# baseline (speedup 1.0000x reference)
import functools

import jax
import jax.numpy as jnp
from jax import lax
from jax.experimental import pallas as pl
from jax.experimental.pallas import tpu as pltpu
from jax.experimental.pallas import tpu_sc as plsc

N = 10000
E = 320000
NC = 2
NS = 16
NW = NC * NS
K = 128
EPAD = 32 * 79 * K
NTRASH = 16
EPT = EPAD // NW
C = EPT // K
NBUF = 3
NITER = (C + 2 * NBUF - 1) // NBUF
KD = 128
CD = EPT // KD
NBUFD = 8
NITERD = (CD + 2 * NBUFD - 1) // NBUFD
RPT = 624
TAIL0 = NS * RPT
TAILN = N - TAIL0
BLK = 1000
NB = N // BLK


def _sc_mesh():
    return plsc.VectorSubcoreMesh(core_axis_name="c", subcore_axis_name="s")


def _make_deg_kernel():
    @functools.partial(
        pl.kernel,
        mesh=_sc_mesh(),
        out_type=jax.ShapeDtypeStruct((NC, N, 128), jnp.float32),
        scratch_types=[
            pltpu.VMEM_SHARED((N + NTRASH, 128), jnp.float32),
            pltpu.VMEM((NBUFD, KD), jnp.int32),
            pltpu.VMEM((KD, 128), jnp.float32),
            pltpu.SemaphoreType.DMA((NBUFD,)),
            pltpu.SemaphoreType.DMA((NBUFD,)),
        ],
    )
    def k(dst_hbm, ones_hbm, z16_hbm, out_hbm, acc, dbuf, obuf, isem, ssem):
        cid = lax.axis_index("c")
        sid = lax.axis_index("s")
        base = (cid * NS + sid) * EPT
        r0 = sid * RPT
        pltpu.sync_copy(ones_hbm, obuf)
        pltpu.sync_copy(z16_hbm.at[pl.ds(r0, RPT)], acc.at[pl.ds(r0, RPT)])

        @pl.when(sid == NS - 1)
        def _():
            pltpu.sync_copy(z16_hbm.at[pl.ds(TAIL0, TAILN)],
                            acc.at[pl.ds(TAIL0, TAILN)])

        plsc.subcore_barrier()

        @pl.loop(0, NITERD)
        def _(co):
            for b in range(NBUFD):
                c = co * NBUFD + b

                @pl.when(jnp.logical_and(c >= NBUFD, c < CD + NBUFD))
                def _():
                    pltpu.make_async_copy(obuf, acc.at[dbuf.at[b]],
                                          ssem.at[b]).wait()

                @pl.when(c < CD)
                def _():
                    pltpu.async_copy(dst_hbm.at[pl.ds(base + c * KD, KD)],
                                     dbuf.at[b], isem.at[b])

                bp = (b - 1) % NBUFD

                @pl.when(jnp.logical_and(c >= 1, c < CD + 1))
                def _():
                    pltpu.make_async_copy(dst_hbm.at[pl.ds(base, KD)],
                                          dbuf.at[bp], isem.at[bp]).wait()
                    pltpu.async_copy(obuf, acc.at[dbuf.at[bp]], ssem.at[bp],
                                     add=True)

        plsc.subcore_barrier()
        pltpu.sync_copy(acc.at[pl.ds(r0, RPT)], out_hbm.at[cid, pl.ds(r0, RPT)])

        @pl.when(sid == NS - 1)
        def _():
            pltpu.sync_copy(acc.at[pl.ds(TAIL0, TAILN)],
                            out_hbm.at[cid, pl.ds(TAIL0, TAILN)])

    return k


def _make_prop_kernel(F):
    @functools.partial(
        pl.kernel,
        mesh=_sc_mesh(),
        out_type=jax.ShapeDtypeStruct((NC, N, F), jnp.float32),
        scratch_types=[
            pltpu.VMEM_SHARED((N + NTRASH, F), jnp.float32),
            pltpu.VMEM((NBUF, K), jnp.int32),
            pltpu.VMEM((NBUF, K), jnp.int32),
            pltpu.VMEM((NBUF, K, F), jnp.float32),
            pltpu.SemaphoreType.DMA((NBUF,)),
            pltpu.SemaphoreType.DMA((NBUF,)),
            pltpu.SemaphoreType.DMA((NBUF,)),
        ],
    )
    def k(hs_hbm, src_hbm, dst_hbm, zf_hbm, out_hbm, acc, sbuf, dbuf, rows,
          isem, gsem, ssem):
        cid = lax.axis_index("c")
        sid = lax.axis_index("s")
        base = (cid * NS + sid) * EPT
        r0 = sid * RPT

        @pl.when(cid == 0)
        def _():
            pltpu.sync_copy(hs_hbm.at[pl.ds(r0, RPT)], acc.at[pl.ds(r0, RPT)])

            @pl.when(sid == NS - 1)
            def _():
                pltpu.sync_copy(hs_hbm.at[pl.ds(TAIL0, TAILN)],
                                acc.at[pl.ds(TAIL0, TAILN)])

        @pl.when(cid == 1)
        def _():
            pltpu.sync_copy(zf_hbm.at[pl.ds(r0, RPT)], acc.at[pl.ds(r0, RPT)])

            @pl.when(sid == NS - 1)
            def _():
                pltpu.sync_copy(zf_hbm.at[pl.ds(TAIL0, TAILN)],
                                acc.at[pl.ds(TAIL0, TAILN)])

        plsc.subcore_barrier()

        @pl.loop(0, NITER)
        def _(co):
            for b in range(NBUF):
                c = co * NBUF + b

                @pl.when(jnp.logical_and(c >= NBUF, c < C + NBUF))
                def _():
                    pltpu.make_async_copy(rows.at[b], acc.at[dbuf.at[b]],
                                          ssem.at[b]).wait()

                @pl.when(c < C)
                def _():
                    e0 = base + c * K
                    pltpu.async_copy(src_hbm.at[pl.ds(e0, K)], sbuf.at[b],
                                     isem.at[b])
                    pltpu.async_copy(dst_hbm.at[pl.ds(e0, K)], dbuf.at[b],
                                     isem.at[b])

                bp = (b - 1) % NBUF

                @pl.when(jnp.logical_and(c >= 1, c < C + 1))
                def _():
                    pltpu.make_async_copy(src_hbm.at[pl.ds(base, K)],
                                          sbuf.at[bp], isem.at[bp]).wait()
                    pltpu.make_async_copy(dst_hbm.at[pl.ds(base, K)],
                                          dbuf.at[bp], isem.at[bp]).wait()
                    pltpu.async_copy(hs_hbm.at[sbuf.at[bp]], rows.at[bp],
                                     gsem.at[bp])

                bq = (b - 2) % NBUF

                @pl.when(jnp.logical_and(c >= 2, c < C + 2))
                def _():
                    pltpu.make_async_copy(hs_hbm.at[sbuf.at[bq]], rows.at[bq],
                                          gsem.at[bq]).wait()
                    pltpu.async_copy(rows.at[bq], acc.at[dbuf.at[bq]],
                                     ssem.at[bq], add=True)

        plsc.subcore_barrier()
        pltpu.sync_copy(acc.at[pl.ds(r0, RPT)], out_hbm.at[cid, pl.ds(r0, RPT)])

        @pl.when(sid == NS - 1)
        def _():
            pltpu.sync_copy(acc.at[pl.ds(TAIL0, TAILN)],
                            out_hbm.at[cid, pl.ds(TAIL0, TAILN)])

    return k


def _dot(a, b):
    return jnp.dot(a, b, preferred_element_type=jnp.float32,
                   precision=lax.Precision.HIGHEST)


def _rsqrt(x):
    r = lax.rsqrt(x)
    return r * (1.5 - 0.5 * x * r * r)


def _prep_body(x_ref, degp_ref, ys_ref, dinv_ref):
    deg = degp_ref[0, :, 0:1] + degp_ref[1, :, 0:1] + 1.0
    dinv = _rsqrt(deg)
    ys_ref[...] = x_ref[...] * dinv
    dinv_ref[...] = dinv


def _tc_prep(x, degp):
    return pl.pallas_call(
        _prep_body,
        grid=(NB,),
        in_specs=[
            pl.BlockSpec((BLK, 128), lambda i: (i, 0)),
            pl.BlockSpec((NC, BLK, 128), lambda i: (0, i, 0)),
        ],
        out_specs=[
            pl.BlockSpec((BLK, 128), lambda i: (i, 0)),
            pl.BlockSpec((BLK, 1), lambda i: (i, 0)),
        ],
        out_shape=[
            jax.ShapeDtypeStruct((N, 128), jnp.float32),
            jax.ShapeDtypeStruct((N, 1), jnp.float32),
        ],
    )(x, degp)


def _merge_body(parts_ref, dinv_ref, w_ref, b_ref, g_ref, be_ref, o_ref,
                stats, mv, *, last, wl0=None, bl0=None, wl1=None):
    phase = pl.program_id(0)
    i = pl.program_id(1)
    dinv = dinv_ref[...]
    t = _dot((parts_ref[0] + parts_ref[1]) * dinv, w_ref[...]) + b_ref[...]

    @pl.when(phase == 0)
    def _():
        @pl.when(i == 0)
        def _():
            stats[...] = jnp.zeros_like(stats)

        stats[0:1] = stats[0:1] + jnp.sum(t, axis=0, keepdims=True)
        stats[1:2] = stats[1:2] + jnp.sum(t * t, axis=0, keepdims=True)

    @pl.when(phase == 1)
    def _():
        @pl.when(i == 0)
        def _():
            mu = stats[0:1] / N
            var = stats[1:2] / N - mu * mu
            mv[0:1] = mu
            mv[1:2] = _rsqrt(var + 1e-5)

        y = g_ref[...] * (t - mv[0:1]) * mv[1:2] + be_ref[...]
        y = jnp.maximum(y, 0.0)
        if last:
            z = jnp.maximum(_dot(y, wl0[...]) + bl0[...], 0.0)
            wb = wl1[...]
            o_ref[...] = _dot(z, wb[0:128]) + wb[128:129]
        else:
            o_ref[...] = y * dinv


def _tc_merge(parts, dinv, W, b, g, be, F):
    body = functools.partial(_merge_body, last=False)
    return pl.pallas_call(
        body,
        grid=(2, NB),
        in_specs=[
            pl.BlockSpec((NC, BLK, 128), lambda p, i: (0, i, 0)),
            pl.BlockSpec((BLK, 1), lambda p, i: (i, 0)),
            pl.BlockSpec((128, F), lambda p, i: (0, 0)),
            pl.BlockSpec((1, F), lambda p, i: (0, 0)),
            pl.BlockSpec((1, F), lambda p, i: (0, 0)),
            pl.BlockSpec((1, F), lambda p, i: (0, 0)),
        ],
        out_specs=pl.BlockSpec((BLK, F), lambda p, i: (i * p, 0)),
        out_shape=jax.ShapeDtypeStruct((N, F), jnp.float32),
        scratch_shapes=[
            pltpu.VMEM((2, F), jnp.float32),
            pltpu.VMEM((2, F), jnp.float32),
        ],
    )(parts, dinv, W, b, g, be)


def _tc_final(parts, dinv, W2, b, g, be, Wl0, bl0, Wl1bl1):
    F = 192

    def wrapped(parts_ref, dinv_ref, w_ref, b_ref, g_ref, be_ref,
                wl0_ref, bl0_ref, wl1_ref, o_ref, stats, mv):
        _merge_body(parts_ref, dinv_ref, w_ref, b_ref, g_ref, be_ref, o_ref,
                    stats, mv, last=True, wl0=wl0_ref, bl0=bl0_ref,
                    wl1=wl1_ref)

    return pl.pallas_call(
        wrapped,
        grid=(2, NB),
        in_specs=[
            pl.BlockSpec((NC, BLK, 128), lambda p, i: (0, i, 0)),
            pl.BlockSpec((BLK, 1), lambda p, i: (i, 0)),
            pl.BlockSpec((128, F), lambda p, i: (0, 0)),
            pl.BlockSpec((1, F), lambda p, i: (0, 0)),
            pl.BlockSpec((1, F), lambda p, i: (0, 0)),
            pl.BlockSpec((1, F), lambda p, i: (0, 0)),
            pl.BlockSpec((F, 128), lambda p, i: (0, 0)),
            pl.BlockSpec((1, 128), lambda p, i: (0, 0)),
            pl.BlockSpec((136, 2), lambda p, i: (0, 0)),
        ],
        out_specs=pl.BlockSpec((BLK, 2), lambda p, i: (i * p, 0)),
        out_shape=jax.ShapeDtypeStruct((N, 2), jnp.float32),
        scratch_shapes=[
            pltpu.VMEM((2, F), jnp.float32),
            pltpu.VMEM((2, F), jnp.float32),
        ],
    )(parts, dinv, W2, b, g, be, Wl0, bl0, Wl1bl1)


def kernel(x, edge_index, W0, b0, W1, b1, W2, b2, g0, be0, g1, be1, g2, be2,
           Wl0, bl0, Wl1, bl1):
    src = edge_index[0].astype(jnp.int32)
    dst = edge_index[1].astype(jnp.int32)
    pad = EPAD - E
    apad = jnp.arange(pad, dtype=jnp.int32)
    src = jnp.concatenate([src, (apad * 997) % N])
    dst = jnp.concatenate([dst, N + (apad % NTRASH)])

    ones_k = jnp.ones((KD, 128), jnp.float32)
    z128 = jnp.zeros((N, 128), jnp.float32)

    degp = _make_deg_kernel()(dst, ones_k, z128)
    ys0, dinv = _tc_prep(x, degp)

    prop = _make_prop_kernel(128)

    s0 = prop(ys0, src, dst, z128)
    ys1 = _tc_merge(s0, dinv, W0, b0.reshape(1, -1), g0.reshape(1, -1),
                    be0.reshape(1, -1), 128)
    s1 = prop(ys1, src, dst, z128)
    ys2 = _tc_merge(s1, dinv, W1, b1.reshape(1, -1), g1.reshape(1, -1),
                    be1.reshape(1, -1), 128)
    s2 = prop(ys2, src, dst, z128)

    wl1b = jnp.concatenate(
        [Wl1, bl1.reshape(1, 2), jnp.zeros((7, 2), jnp.float32)], axis=0)
    out = _tc_final(s2, dinv, W2, b2.reshape(1, -1), g2.reshape(1, -1),
                    be2.reshape(1, -1), Wl0, bl0.reshape(1, -1), wl1b)
    return out


# speedup vs baseline: 1.0320x; 1.0320x over previous
import functools

import jax
import jax.numpy as jnp
from jax import lax
from jax.experimental import pallas as pl
from jax.experimental.pallas import tpu as pltpu
from jax.experimental.pallas import tpu_sc as plsc

N = 10000
E = 320000
NC = 2
NS = 16
NW = NC * NS
EPT = E // NW
K = 80
C = EPT // K
NBUF = 4
NITER = (C + 2 * NBUF - 1) // NBUF
KD = 80
CD = EPT // KD
NBUFD = 12
NITERD = (CD + 2 * NBUFD - 1) // NBUFD
RPT = 624
TAIL0 = NS * RPT
TAILN = N - TAIL0
BLK = 1000
NB = N // BLK


def _sc_mesh():
    return plsc.VectorSubcoreMesh(core_axis_name="c", subcore_axis_name="s")


def _make_deg_kernel():
    @functools.partial(
        pl.kernel,
        mesh=_sc_mesh(),
        out_type=jax.ShapeDtypeStruct((NC, N, 128), jnp.float32),
        scratch_types=[
            pltpu.VMEM_SHARED((N, 128), jnp.float32),
            pltpu.VMEM((NBUFD, KD), jnp.int32),
            pltpu.VMEM((KD, 128), jnp.float32),
            pltpu.SemaphoreType.DMA((NBUFD,)),
            pltpu.SemaphoreType.DMA((NBUFD,)),
        ],
    )
    def k(dst_hbm, ones_hbm, z16_hbm, out_hbm, acc, dbuf, obuf, isem, ssem):
        cid = lax.axis_index("c")
        sid = lax.axis_index("s")
        base = (cid * NS + sid) * EPT
        r0 = sid * RPT
        pltpu.sync_copy(ones_hbm, obuf)
        pltpu.sync_copy(z16_hbm.at[pl.ds(r0, RPT)], acc.at[pl.ds(r0, RPT)])

        @pl.when(sid == NS - 1)
        def _():
            pltpu.sync_copy(z16_hbm.at[pl.ds(TAIL0, TAILN)],
                            acc.at[pl.ds(TAIL0, TAILN)])

        plsc.subcore_barrier()

        @pl.loop(0, NITERD)
        def _(co):
            for b in range(NBUFD):
                c = co * NBUFD + b

                @pl.when(jnp.logical_and(c >= NBUFD, c < CD + NBUFD))
                def _():
                    pltpu.make_async_copy(obuf, acc.at[dbuf.at[b]],
                                          ssem.at[b]).wait()

                @pl.when(c < CD)
                def _():
                    pltpu.async_copy(dst_hbm.at[pl.ds(base + c * KD, KD)],
                                     dbuf.at[b], isem.at[b])

                bp = (b - 1) % NBUFD

                @pl.when(jnp.logical_and(c >= 1, c < CD + 1))
                def _():
                    pltpu.make_async_copy(dst_hbm.at[pl.ds(base, KD)],
                                          dbuf.at[bp], isem.at[bp]).wait()
                    pltpu.async_copy(obuf, acc.at[dbuf.at[bp]], ssem.at[bp],
                                     add=True)

        plsc.subcore_barrier()
        pltpu.sync_copy(acc.at[pl.ds(r0, RPT)], out_hbm.at[cid, pl.ds(r0, RPT)])

        @pl.when(sid == NS - 1)
        def _():
            pltpu.sync_copy(acc.at[pl.ds(TAIL0, TAILN)],
                            out_hbm.at[cid, pl.ds(TAIL0, TAILN)])

    return k


def _make_prop_kernel(F):
    @functools.partial(
        pl.kernel,
        mesh=_sc_mesh(),
        out_type=jax.ShapeDtypeStruct((NC, N, F), jnp.float32),
        scratch_types=[
            pltpu.VMEM_SHARED((N, F), jnp.float32),
            pltpu.VMEM((NBUF, K), jnp.int32),
            pltpu.VMEM((NBUF, K), jnp.int32),
            pltpu.VMEM((NBUF, K, F), jnp.float32),
            pltpu.SemaphoreType.DMA((NBUF,)),
            pltpu.SemaphoreType.DMA((NBUF,)),
            pltpu.SemaphoreType.DMA((NBUF,)),
        ],
    )
    def k(hs_hbm, src_hbm, dst_hbm, zf_hbm, out_hbm, acc, sbuf, dbuf, rows,
          isem, gsem, ssem):
        cid = lax.axis_index("c")
        sid = lax.axis_index("s")
        base = (cid * NS + sid) * EPT
        r0 = sid * RPT

        @pl.when(cid == 0)
        def _():
            pltpu.sync_copy(hs_hbm.at[pl.ds(r0, RPT)], acc.at[pl.ds(r0, RPT)])

            @pl.when(sid == NS - 1)
            def _():
                pltpu.sync_copy(hs_hbm.at[pl.ds(TAIL0, TAILN)],
                                acc.at[pl.ds(TAIL0, TAILN)])

        @pl.when(cid == 1)
        def _():
            pltpu.sync_copy(zf_hbm.at[pl.ds(r0, RPT)], acc.at[pl.ds(r0, RPT)])

            @pl.when(sid == NS - 1)
            def _():
                pltpu.sync_copy(zf_hbm.at[pl.ds(TAIL0, TAILN)],
                                acc.at[pl.ds(TAIL0, TAILN)])

        plsc.subcore_barrier()

        @pl.loop(0, NITER)
        def _(co):
            for b in range(NBUF):
                c = co * NBUF + b

                @pl.when(jnp.logical_and(c >= NBUF, c < C + NBUF))
                def _():
                    pltpu.make_async_copy(rows.at[b], acc.at[dbuf.at[b]],
                                          ssem.at[b]).wait()

                @pl.when(c < C)
                def _():
                    e0 = base + c * K
                    pltpu.async_copy(src_hbm.at[pl.ds(e0, K)], sbuf.at[b],
                                     isem.at[b])
                    pltpu.async_copy(dst_hbm.at[pl.ds(e0, K)], dbuf.at[b],
                                     isem.at[b])

                bp = (b - 1) % NBUF

                @pl.when(jnp.logical_and(c >= 1, c < C + 1))
                def _():
                    pltpu.make_async_copy(src_hbm.at[pl.ds(base, K)],
                                          sbuf.at[bp], isem.at[bp]).wait()
                    pltpu.make_async_copy(dst_hbm.at[pl.ds(base, K)],
                                          dbuf.at[bp], isem.at[bp]).wait()
                    pltpu.async_copy(hs_hbm.at[sbuf.at[bp]], rows.at[bp],
                                     gsem.at[bp])

                bq = (b - 2) % NBUF

                @pl.when(jnp.logical_and(c >= 2, c < C + 2))
                def _():
                    pltpu.make_async_copy(hs_hbm.at[sbuf.at[bq]], rows.at[bq],
                                          gsem.at[bq]).wait()
                    pltpu.async_copy(rows.at[bq], acc.at[dbuf.at[bq]],
                                     ssem.at[bq], add=True)

        plsc.subcore_barrier()
        pltpu.sync_copy(acc.at[pl.ds(r0, RPT)], out_hbm.at[cid, pl.ds(r0, RPT)])

        @pl.when(sid == NS - 1)
        def _():
            pltpu.sync_copy(acc.at[pl.ds(TAIL0, TAILN)],
                            out_hbm.at[cid, pl.ds(TAIL0, TAILN)])

    return k


def _dot(a, b):
    return jnp.dot(a, b, preferred_element_type=jnp.float32,
                   precision=lax.Precision.HIGHEST)


def _rsqrt(x):
    r = lax.rsqrt(x)
    return r * (1.5 - 0.5 * x * r * r)


def _prep_body(x_ref, degp_ref, ys_ref, dinv_ref):
    deg = degp_ref[0, :, 0:1] + degp_ref[1, :, 0:1] + 1.0
    dinv = _rsqrt(deg)
    ys_ref[...] = x_ref[...] * dinv
    dinv_ref[...] = dinv


def _tc_prep(x, degp):
    return pl.pallas_call(
        _prep_body,
        grid=(NB,),
        in_specs=[
            pl.BlockSpec((BLK, 128), lambda i: (i, 0)),
            pl.BlockSpec((NC, BLK, 128), lambda i: (0, i, 0)),
        ],
        out_specs=[
            pl.BlockSpec((BLK, 128), lambda i: (i, 0)),
            pl.BlockSpec((BLK, 1), lambda i: (i, 0)),
        ],
        out_shape=[
            jax.ShapeDtypeStruct((N, 128), jnp.float32),
            jax.ShapeDtypeStruct((N, 1), jnp.float32),
        ],
    )(x, degp)


def _merge_body(parts_ref, dinv_ref, w_ref, b_ref, g_ref, be_ref, o_ref,
                stats, mv, *, last, wl0=None, bl0=None, wl1=None):
    phase = pl.program_id(0)
    i = pl.program_id(1)
    dinv = dinv_ref[...]
    t = _dot((parts_ref[0] + parts_ref[1]) * dinv, w_ref[...]) + b_ref[...]

    @pl.when(phase == 0)
    def _():
        @pl.when(i == 0)
        def _():
            stats[...] = jnp.zeros_like(stats)

        stats[0:1] = stats[0:1] + jnp.sum(t, axis=0, keepdims=True)
        stats[1:2] = stats[1:2] + jnp.sum(t * t, axis=0, keepdims=True)

    @pl.when(phase == 1)
    def _():
        @pl.when(i == 0)
        def _():
            mu = stats[0:1] / N
            var = stats[1:2] / N - mu * mu
            mv[0:1] = mu
            mv[1:2] = _rsqrt(var + 1e-5)

        y = g_ref[...] * (t - mv[0:1]) * mv[1:2] + be_ref[...]
        y = jnp.maximum(y, 0.0)
        if last:
            z = jnp.maximum(_dot(y, wl0[...]) + bl0[...], 0.0)
            wb = wl1[...]
            o_ref[...] = _dot(z, wb[0:128]) + wb[128:129]
        else:
            o_ref[...] = y * dinv


def _tc_merge(parts, dinv, W, b, g, be, F):
    body = functools.partial(_merge_body, last=False)
    return pl.pallas_call(
        body,
        grid=(2, NB),
        in_specs=[
            pl.BlockSpec((NC, BLK, 128), lambda p, i: (0, i, 0)),
            pl.BlockSpec((BLK, 1), lambda p, i: (i, 0)),
            pl.BlockSpec((128, F), lambda p, i: (0, 0)),
            pl.BlockSpec((1, F), lambda p, i: (0, 0)),
            pl.BlockSpec((1, F), lambda p, i: (0, 0)),
            pl.BlockSpec((1, F), lambda p, i: (0, 0)),
        ],
        out_specs=pl.BlockSpec((BLK, F), lambda p, i: (i * p, 0)),
        out_shape=jax.ShapeDtypeStruct((N, F), jnp.float32),
        scratch_shapes=[
            pltpu.VMEM((2, F), jnp.float32),
            pltpu.VMEM((2, F), jnp.float32),
        ],
    )(parts, dinv, W, b, g, be)


def _tc_final(parts, dinv, W2, b, g, be, Wl0, bl0, Wl1bl1):
    F = 192

    def wrapped(parts_ref, dinv_ref, w_ref, b_ref, g_ref, be_ref,
                wl0_ref, bl0_ref, wl1_ref, o_ref, stats, mv):
        _merge_body(parts_ref, dinv_ref, w_ref, b_ref, g_ref, be_ref, o_ref,
                    stats, mv, last=True, wl0=wl0_ref, bl0=bl0_ref,
                    wl1=wl1_ref)

    return pl.pallas_call(
        wrapped,
        grid=(2, NB),
        in_specs=[
            pl.BlockSpec((NC, BLK, 128), lambda p, i: (0, i, 0)),
            pl.BlockSpec((BLK, 1), lambda p, i: (i, 0)),
            pl.BlockSpec((128, F), lambda p, i: (0, 0)),
            pl.BlockSpec((1, F), lambda p, i: (0, 0)),
            pl.BlockSpec((1, F), lambda p, i: (0, 0)),
            pl.BlockSpec((1, F), lambda p, i: (0, 0)),
            pl.BlockSpec((F, 128), lambda p, i: (0, 0)),
            pl.BlockSpec((1, 128), lambda p, i: (0, 0)),
            pl.BlockSpec((136, 2), lambda p, i: (0, 0)),
        ],
        out_specs=pl.BlockSpec((BLK, 2), lambda p, i: (i * p, 0)),
        out_shape=jax.ShapeDtypeStruct((N, 2), jnp.float32),
        scratch_shapes=[
            pltpu.VMEM((2, F), jnp.float32),
            pltpu.VMEM((2, F), jnp.float32),
        ],
    )(parts, dinv, W2, b, g, be, Wl0, bl0, Wl1bl1)


def kernel(x, edge_index, W0, b0, W1, b1, W2, b2, g0, be0, g1, be1, g2, be2,
           Wl0, bl0, Wl1, bl1):
    src = edge_index[0].astype(jnp.int32)
    dst = edge_index[1].astype(jnp.int32)

    ones_k = jnp.ones((KD, 128), jnp.float32)
    z128 = jnp.zeros((N, 128), jnp.float32)

    degp = _make_deg_kernel()(dst, ones_k, z128)
    ys0, dinv = _tc_prep(x, degp)

    prop = _make_prop_kernel(128)

    s0 = prop(ys0, src, dst, z128)
    ys1 = _tc_merge(s0, dinv, W0, b0.reshape(1, -1), g0.reshape(1, -1),
                    be0.reshape(1, -1), 128)
    s1 = prop(ys1, src, dst, z128)
    ys2 = _tc_merge(s1, dinv, W1, b1.reshape(1, -1), g1.reshape(1, -1),
                    be1.reshape(1, -1), 128)
    s2 = prop(ys2, src, dst, z128)

    wl1b = jnp.concatenate(
        [Wl1, bl1.reshape(1, 2), jnp.zeros((7, 2), jnp.float32)], axis=0)
    out = _tc_final(s2, dinv, W2, b2.reshape(1, -1), g2.reshape(1, -1),
                    be2.reshape(1, -1), Wl0, bl0.reshape(1, -1), wl1b)
    return out


# speedup vs baseline: 1.0752x; 1.0418x over previous
import functools

import jax
import jax.numpy as jnp
from jax import lax
from jax.experimental import pallas as pl
from jax.experimental.pallas import tpu as pltpu
from jax.experimental.pallas import tpu_sc as plsc

N = 10000
E = 320000
NC = 2
NS = 16
NW = NC * NS
EPT = E // NW
K = 80
C = EPT // K
NBUF = 4
NITER = (C + 2 * NBUF - 1) // NBUF
KD = 80
CD = EPT // KD
NBUFD = 12
NITERD = (CD + 2 * NBUFD - 1) // NBUFD
RPT = 624
TAIL0 = NS * RPT
TAILN = N - TAIL0
BLK = 1000
NB = N // BLK


def _sc_mesh():
    return plsc.VectorSubcoreMesh(core_axis_name="c", subcore_axis_name="s")


def _make_deg_kernel():
    @functools.partial(
        pl.kernel,
        mesh=_sc_mesh(),
        out_type=jax.ShapeDtypeStruct((NC, N, 128), jnp.float32),
        scratch_types=[
            pltpu.VMEM_SHARED((N, 128), jnp.float32),
            pltpu.VMEM((NBUFD, KD), jnp.int32),
            pltpu.VMEM((KD, 128), jnp.float32),
            pltpu.SemaphoreType.DMA((NBUFD,)),
            pltpu.SemaphoreType.DMA((NBUFD,)),
        ],
    )
    def k(dst_hbm, ones_hbm, z16_hbm, out_hbm, acc, dbuf, obuf, isem, ssem):
        cid = lax.axis_index("c")
        sid = lax.axis_index("s")
        base = (cid * NS + sid) * EPT
        r0 = sid * RPT
        pltpu.sync_copy(ones_hbm, obuf)
        pltpu.sync_copy(z16_hbm.at[pl.ds(r0, RPT)], acc.at[pl.ds(r0, RPT)])

        @pl.when(sid == NS - 1)
        def _():
            pltpu.sync_copy(z16_hbm.at[pl.ds(TAIL0, TAILN)],
                            acc.at[pl.ds(TAIL0, TAILN)])

        plsc.subcore_barrier()

        @pl.loop(0, NITERD)
        def _(co):
            for b in range(NBUFD):
                c = co * NBUFD + b

                @pl.when(jnp.logical_and(c >= NBUFD, c < CD + NBUFD))
                def _():
                    pltpu.make_async_copy(obuf, acc.at[dbuf.at[b]],
                                          ssem.at[b]).wait()

                @pl.when(c < CD)
                def _():
                    pltpu.async_copy(dst_hbm.at[pl.ds(base + c * KD, KD)],
                                     dbuf.at[b], isem.at[b])

                bp = (b - 1) % NBUFD

                @pl.when(jnp.logical_and(c >= 1, c < CD + 1))
                def _():
                    pltpu.make_async_copy(dst_hbm.at[pl.ds(base, KD)],
                                          dbuf.at[bp], isem.at[bp]).wait()
                    pltpu.async_copy(obuf, acc.at[dbuf.at[bp]], ssem.at[bp],
                                     add=True)

        plsc.subcore_barrier()
        pltpu.sync_copy(acc.at[pl.ds(r0, RPT)], out_hbm.at[cid, pl.ds(r0, RPT)])

        @pl.when(sid == NS - 1)
        def _():
            pltpu.sync_copy(acc.at[pl.ds(TAIL0, TAILN)],
                            out_hbm.at[cid, pl.ds(TAIL0, TAILN)])

    return k


def _make_prop_kernel(F):
    @functools.partial(
        pl.kernel,
        mesh=_sc_mesh(),
        out_type=jax.ShapeDtypeStruct((NC, N, F), jnp.float32),
        scratch_types=[
            pltpu.VMEM_SHARED((N, F), jnp.float32),
            pltpu.VMEM((NBUF, K), jnp.int32),
            pltpu.VMEM((NBUF, K), jnp.int32),
            pltpu.VMEM((NBUF, K, F), jnp.float32),
            pltpu.SemaphoreType.DMA((NBUF,)),
            pltpu.SemaphoreType.DMA((NBUF,)),
            pltpu.SemaphoreType.DMA((NBUF,)),
        ],
    )
    def k(hs_hbm, src_hbm, dst_hbm, zf_hbm, out_hbm, acc, sbuf, dbuf, rows,
          isem, gsem, ssem):
        cid = lax.axis_index("c")
        sid = lax.axis_index("s")
        base = (cid * NS + sid) * EPT
        r0 = sid * RPT

        @pl.when(cid == 0)
        def _():
            pltpu.sync_copy(hs_hbm.at[pl.ds(r0, RPT)], acc.at[pl.ds(r0, RPT)])

            @pl.when(sid == NS - 1)
            def _():
                pltpu.sync_copy(hs_hbm.at[pl.ds(TAIL0, TAILN)],
                                acc.at[pl.ds(TAIL0, TAILN)])

        @pl.when(cid == 1)
        def _():
            pltpu.sync_copy(zf_hbm.at[pl.ds(r0, RPT)], acc.at[pl.ds(r0, RPT)])

            @pl.when(sid == NS - 1)
            def _():
                pltpu.sync_copy(zf_hbm.at[pl.ds(TAIL0, TAILN)],
                                acc.at[pl.ds(TAIL0, TAILN)])

        plsc.subcore_barrier()

        @pl.loop(0, NITER)
        def _(co):
            for b in range(NBUF):
                c = co * NBUF + b

                @pl.when(jnp.logical_and(c >= NBUF, c < C + NBUF))
                def _():
                    pltpu.make_async_copy(rows.at[b], acc.at[dbuf.at[b]],
                                          ssem.at[b]).wait()

                @pl.when(c < C)
                def _():
                    e0 = base + c * K
                    pltpu.async_copy(src_hbm.at[pl.ds(e0, K)], sbuf.at[b],
                                     isem.at[b])
                    pltpu.async_copy(dst_hbm.at[pl.ds(e0, K)], dbuf.at[b],
                                     isem.at[b])

                bp = (b - 1) % NBUF

                @pl.when(jnp.logical_and(c >= 1, c < C + 1))
                def _():
                    pltpu.make_async_copy(src_hbm.at[pl.ds(base, K)],
                                          sbuf.at[bp], isem.at[bp]).wait()
                    pltpu.make_async_copy(dst_hbm.at[pl.ds(base, K)],
                                          dbuf.at[bp], isem.at[bp]).wait()
                    pltpu.async_copy(hs_hbm.at[sbuf.at[bp]], rows.at[bp],
                                     gsem.at[bp])

                bq = (b - 2) % NBUF

                @pl.when(jnp.logical_and(c >= 2, c < C + 2))
                def _():
                    pltpu.make_async_copy(hs_hbm.at[sbuf.at[bq]], rows.at[bq],
                                          gsem.at[bq]).wait()
                    pltpu.async_copy(rows.at[bq], acc.at[dbuf.at[bq]],
                                     ssem.at[bq], add=True)

        plsc.subcore_barrier()
        pltpu.sync_copy(acc.at[pl.ds(r0, RPT)], out_hbm.at[cid, pl.ds(r0, RPT)])

        @pl.when(sid == NS - 1)
        def _():
            pltpu.sync_copy(acc.at[pl.ds(TAIL0, TAILN)],
                            out_hbm.at[cid, pl.ds(TAIL0, TAILN)])

    return k


def _dot(a, b):
    return jnp.dot(a, b, preferred_element_type=jnp.float32,
                   precision=lax.Precision.HIGHEST)


def _rsqrt(x):
    r = lax.rsqrt(x)
    return r * (1.5 - 0.5 * x * r * r)


def _prep_body(x_ref, degp_ref, ys_ref, dinv_ref):
    deg = degp_ref[0, :, 0:1] + degp_ref[1, :, 0:1] + 1.0
    dinv = _rsqrt(deg)
    ys_ref[...] = x_ref[...] * dinv
    dinv_ref[...] = dinv


def _tc_prep(x, degp):
    return pl.pallas_call(
        _prep_body,
        grid=(NB,),
        in_specs=[
            pl.BlockSpec((BLK, 128), lambda i: (i, 0)),
            pl.BlockSpec((NC, BLK, 128), lambda i: (0, i, 0)),
        ],
        out_specs=[
            pl.BlockSpec((BLK, 128), lambda i: (i, 0)),
            pl.BlockSpec((BLK, 1), lambda i: (i, 0)),
        ],
        out_shape=[
            jax.ShapeDtypeStruct((N, 128), jnp.float32),
            jax.ShapeDtypeStruct((N, 1), jnp.float32),
        ],
    )(x, degp)


def _merge_body(parts_ref, dinv_ref, w_ref, b_ref, g_ref, be_ref, o_ref,
                stats, mv, tbuf, *, last, wl0=None, bl0=None, wl1=None):
    phase = pl.program_id(0)
    i = pl.program_id(1)
    dinv = dinv_ref[...]

    @pl.when(phase == 0)
    def _():
        t = _dot((parts_ref[0] + parts_ref[1]) * dinv, w_ref[...]) + b_ref[...]
        tbuf[pl.ds(i * BLK, BLK)] = t

        @pl.when(i == 0)
        def _():
            stats[...] = jnp.zeros_like(stats)

        stats[0:1] = stats[0:1] + jnp.sum(t, axis=0, keepdims=True)
        stats[1:2] = stats[1:2] + jnp.sum(t * t, axis=0, keepdims=True)

    @pl.when(phase == 1)
    def _():
        @pl.when(i == 0)
        def _():
            mu = stats[0:1] / N
            var = stats[1:2] / N - mu * mu
            mv[0:1] = mu
            mv[1:2] = _rsqrt(var + 1e-5)

        t = tbuf[pl.ds(i * BLK, BLK)]
        y = g_ref[...] * (t - mv[0:1]) * mv[1:2] + be_ref[...]
        y = jnp.maximum(y, 0.0)
        if last:
            z = jnp.maximum(_dot(y, wl0[...]) + bl0[...], 0.0)
            wb = wl1[...]
            o_ref[...] = _dot(z, wb[0:128]) + wb[128:129]
        else:
            o_ref[...] = y * dinv


def _tc_merge(parts, dinv, W, b, g, be, F):
    body = functools.partial(_merge_body, last=False)
    return pl.pallas_call(
        body,
        grid=(2, NB),
        in_specs=[
            pl.BlockSpec((NC, BLK, 128), lambda p, i: (0, i * (1 - p), 0)),
            pl.BlockSpec((BLK, 1), lambda p, i: (i, 0)),
            pl.BlockSpec((128, F), lambda p, i: (0, 0)),
            pl.BlockSpec((1, F), lambda p, i: (0, 0)),
            pl.BlockSpec((1, F), lambda p, i: (0, 0)),
            pl.BlockSpec((1, F), lambda p, i: (0, 0)),
        ],
        out_specs=pl.BlockSpec((BLK, F), lambda p, i: (i * p, 0)),
        out_shape=jax.ShapeDtypeStruct((N, F), jnp.float32),
        scratch_shapes=[
            pltpu.VMEM((2, F), jnp.float32),
            pltpu.VMEM((2, F), jnp.float32),
            pltpu.VMEM((N, F), jnp.float32),
        ],
    )(parts, dinv, W, b, g, be)


def _tc_final(parts, dinv, W2, b, g, be, Wl0, bl0, Wl1bl1):
    F = 192

    def wrapped(parts_ref, dinv_ref, w_ref, b_ref, g_ref, be_ref,
                wl0_ref, bl0_ref, wl1_ref, o_ref, stats, mv, tbuf):
        _merge_body(parts_ref, dinv_ref, w_ref, b_ref, g_ref, be_ref, o_ref,
                    stats, mv, tbuf, last=True, wl0=wl0_ref, bl0=bl0_ref,
                    wl1=wl1_ref)

    return pl.pallas_call(
        wrapped,
        grid=(2, NB),
        in_specs=[
            pl.BlockSpec((NC, BLK, 128), lambda p, i: (0, i * (1 - p), 0)),
            pl.BlockSpec((BLK, 1), lambda p, i: (i, 0)),
            pl.BlockSpec((128, F), lambda p, i: (0, 0)),
            pl.BlockSpec((1, F), lambda p, i: (0, 0)),
            pl.BlockSpec((1, F), lambda p, i: (0, 0)),
            pl.BlockSpec((1, F), lambda p, i: (0, 0)),
            pl.BlockSpec((F, 128), lambda p, i: (0, 0)),
            pl.BlockSpec((1, 128), lambda p, i: (0, 0)),
            pl.BlockSpec((136, 2), lambda p, i: (0, 0)),
        ],
        out_specs=pl.BlockSpec((BLK, 2), lambda p, i: (i * p, 0)),
        out_shape=jax.ShapeDtypeStruct((N, 2), jnp.float32),
        scratch_shapes=[
            pltpu.VMEM((2, F), jnp.float32),
            pltpu.VMEM((2, F), jnp.float32),
            pltpu.VMEM((N, F), jnp.float32),
        ],
    )(parts, dinv, W2, b, g, be, Wl0, bl0, Wl1bl1)


def kernel(x, edge_index, W0, b0, W1, b1, W2, b2, g0, be0, g1, be1, g2, be2,
           Wl0, bl0, Wl1, bl1):
    src = edge_index[0].astype(jnp.int32)
    dst = edge_index[1].astype(jnp.int32)

    ones_k = jnp.ones((KD, 128), jnp.float32)
    z128 = jnp.zeros((N, 128), jnp.float32)

    degp = _make_deg_kernel()(dst, ones_k, z128)
    ys0, dinv = _tc_prep(x, degp)

    prop = _make_prop_kernel(128)

    s0 = prop(ys0, src, dst, z128)
    ys1 = _tc_merge(s0, dinv, W0, b0.reshape(1, -1), g0.reshape(1, -1),
                    be0.reshape(1, -1), 128)
    s1 = prop(ys1, src, dst, z128)
    ys2 = _tc_merge(s1, dinv, W1, b1.reshape(1, -1), g1.reshape(1, -1),
                    be1.reshape(1, -1), 128)
    s2 = prop(ys2, src, dst, z128)

    wl1b = jnp.concatenate(
        [Wl1, bl1.reshape(1, 2), jnp.zeros((7, 2), jnp.float32)], axis=0)
    out = _tc_final(s2, dinv, W2, b2.reshape(1, -1), g2.reshape(1, -1),
                    be2.reshape(1, -1), Wl0, bl0.reshape(1, -1), wl1b)
    return out
